# baseline (speedup 1.0000x reference)
import functools

import jax
import jax.numpy as jnp
from jax.experimental import pallas as pl
from jax.experimental.pallas import tpu as pltpu

_SIGMA = 0.5


def _spec2(shape):
    return pl.BlockSpec(shape, lambda i: (0, 0))


def _fused_kernel(
    permi_ref,
    e1b_ref, e2b_ref, e3b_ref, zb_ref,
    d1b_ref, d2b_ref, d3b_ref, xbb_ref,
    g5w_ref, clu_ref,
    x_hbm, adj_hbm, e1w_hbm, e2w_hbm, e3w_hbm, zw_hbm,
    d1w_hbm, d2w_hbm, d3w_hbm, xbw_hbm,
    g1w_hbm, g2w_hbm, g3w_hbm, g4w_hbm,
    h_hbm, pred_hbm, q_hbm, z_hbm, outh_hbm, negh_hbm, xbar_hbm, summ_hbm,
    sa, sb, sc, sd, se, sf, sg, adj_s, x_s,
    big_o, zo_s, ho_s, po_s, qo_s, su_s,
    sem_a, sem_b, sem_c, sem_d, sem_e, sem_f, sem_g, sem_adj, sem_x,
    sem_o1, sem_o2,
    *, v,
):
    f32 = jnp.float32
    bf16 = jnp.bfloat16
    s = _SIGMA
    n = x_s.shape[0]
    pid = pl.program_id(0)

    def start(src, dst, sem):
        pltpu.make_async_copy(src, dst, sem).start()

    def wait(src, dst, sem):
        pltpu.make_async_copy(src, dst, sem).wait()

    def lin(xv, w_ref, b_ref, relu):
        y = jnp.dot(xv.astype(bf16), w_ref[...].astype(bf16),
                    preferred_element_type=f32)
        y = y + b_ref[...]
        return jnp.maximum(y, 0.0) if relu else y

    def mm(xv, w_ref):
        return jnp.dot(xv.astype(bf16), w_ref[...].astype(bf16),
                       preferred_element_type=f32)

    @pl.when(pid == 0)
    def _pos():
        start(e1w_hbm, sa, sem_a)
        start(x_hbm, x_s, sem_x)
        start(e2w_hbm, sb, sem_b)
        start(e3w_hbm, sc, sem_c)
        start(zw_hbm, sd, sem_d)
        start(adj_hbm, adj_s, sem_adj)
        start(d1w_hbm, se, sem_e)
        start(d2w_hbm, sf, sem_f)
        start(xbw_hbm, sg, sem_g)

        wait(x_hbm, x_s, sem_x)
        xb = x_s[...].astype(bf16)

        wait(e1w_hbm, sa, sem_a)
        t1 = lin(xb, sa, e1b_ref, True)
        start(g1w_hbm, sa, sem_a)

        wait(e2w_hbm, sb, sem_b)
        t2 = lin(t1, sb, e2b_ref, True)
        start(g2w_hbm, sb, sem_b)

        wait(e3w_hbm, sc, sem_c)
        t3 = lin(t2, sc, e3b_ref, True)
        start(g3w_hbm, sc, sem_c)

        wait(zw_hbm, sd, sem_d)
        z = lin(t3, sd, zb_ref, False)
        start(g4w_hbm, sd, sem_d)

        zo_s[...] = z
        start(zo_s, z_hbm, sem_o1)

        wait(adj_hbm, adj_s, sem_adj)
        adjb = adj_s[...].astype(bf16)

        def agg(sup, relu):
            y = jnp.dot(adjb, sup.astype(bf16), preferred_element_type=f32)
            return jnp.maximum(y, 0.0) if relu else y

        wait(g1w_hbm, sa, sem_a)
        h1 = agg(mm(xb, sa), True)

        wait(g2w_hbm, sb, sem_b)
        h2 = agg(mm((1.0 - s) * h1 + s * t1, sb), True)
        start(d3w_hbm, sb, sem_b)

        wait(g3w_hbm, sc, sem_c)
        h3 = agg(mm((1.0 - s) * h2 + s * t2, sc), True)
        out_h = (1.0 - s) * h3 + s * t3
        big_o[...] = out_h
        start(big_o, outh_hbm, sem_o2)

        wait(g4w_hbm, sd, sem_d)
        h4 = agg(mm(out_h, sd), True)
        h5 = agg(jnp.dot(((1.0 - s) * h4 + s * z).astype(bf16),
                         g5w_ref[...].astype(bf16),
                         preferred_element_type=f32), False)

        m = jnp.max(h5, axis=1, keepdims=True)
        e = jnp.exp(h5 - m)
        pred = e * pl.reciprocal(jnp.sum(e, axis=1, keepdims=True),
                                 approx=True)

        mean = jnp.mean(out_h, axis=0, keepdims=True)
        en = jnp.exp(-jnp.abs(mean))
        r = pl.reciprocal(1.0 + en, approx=True)
        su_s[...] = jnp.where(mean >= 0.0, r, en * r)

        c = clu_ref[...]
        z2 = jnp.sum(z * z, axis=1, keepdims=True)
        c2 = jnp.sum(c * c, axis=1, keepdims=True).T
        zc = jnp.dot(z, c.T, preferred_element_type=f32)
        dist = jnp.maximum(z2 + c2 - 2.0 * zc, 0.0)
        q = pl.reciprocal(1.0 + dist * (1.0 / v), approx=True)
        if v != 1.0:
            q = q ** ((v + 1.0) / 2.0)
        q = q * pl.reciprocal(jnp.sum(q, axis=1, keepdims=True), approx=True)

        ho_s[...] = h5
        po_s[...] = pred
        qo_s[...] = q
        start(ho_s, h_hbm, sem_o1)
        start(po_s, pred_hbm, sem_o1)
        start(qo_s, q_hbm, sem_o1)
        start(su_s, summ_hbm, sem_o1)

        wait(d1w_hbm, se, sem_e)
        d1 = lin(z, se, d1b_ref, True)
        wait(d2w_hbm, sf, sem_f)
        d2 = lin(d1, sf, d2b_ref, True)
        wait(d3w_hbm, sb, sem_b)
        d3 = lin(d2, sb, d3b_ref, True)
        wait(xbw_hbm, sg, sem_g)
        x_s[...] = lin(d3, sg, xbb_ref, False)
        start(x_s, xbar_hbm, sem_x)

        wait(zo_s, z_hbm, sem_o1)
        wait(big_o, outh_hbm, sem_o2)
        wait(ho_s, h_hbm, sem_o1)
        wait(po_s, pred_hbm, sem_o1)
        wait(qo_s, q_hbm, sem_o1)
        wait(su_s, summ_hbm, sem_o1)
        wait(x_s, xbar_hbm, sem_x)

    @pl.when(pid == 1)
    def _neg():
        start(e1w_hbm, sa, sem_a)
        start(x_hbm, x_s, sem_x)
        start(e2w_hbm, sb, sem_b)
        start(e3w_hbm, sc, sem_c)
        start(adj_hbm, adj_s, sem_adj)

        lane = jax.lax.broadcasted_iota(jnp.int32, (n, n), 1)
        pmat = (lane == permi_ref[...]).astype(bf16)

        wait(x_hbm, x_s, sem_x)
        xb = x_s[...].astype(bf16)

        wait(e1w_hbm, sa, sem_a)
        t1 = lin(xb, sa, e1b_ref, True)
        start(g1w_hbm, sa, sem_a)

        wait(e2w_hbm, sb, sem_b)
        t2 = lin(t1, sb, e2b_ref, True)
        start(g2w_hbm, sb, sem_b)

        wait(e3w_hbm, sc, sem_c)
        t3 = lin(t2, sc, e3b_ref, True)
        start(g3w_hbm, sc, sem_c)

        tcat = jnp.concatenate(
            [t1.astype(bf16), t2.astype(bf16), t3.astype(bf16)], axis=1)
        ncat = jnp.dot(pmat, tcat, preferred_element_type=f32).astype(bf16)
        c1 = t1.shape[1]
        c2 = c1 + t2.shape[1]
        nt1 = ncat[:, :c1]
        nt2 = ncat[:, c1:c2]
        nt3 = ncat[:, c2:]

        wait(adj_hbm, adj_s, sem_adj)
        adjb = adj_s[...].astype(bf16)

        def agg(sup, relu):
            y = jnp.dot(adjb, sup.astype(bf16), preferred_element_type=f32)
            return jnp.maximum(y, 0.0) if relu else y

        wait(g1w_hbm, sa, sem_a)
        h1 = agg(mm(xb, sa), True)

        wait(g2w_hbm, sb, sem_b)
        nh2 = agg(mm((1.0 - s) * h1 + s * nt1, sb), True)

        wait(g3w_hbm, sc, sem_c)
        nh3 = agg(mm((1.0 - s) * nh2 + s * nt2, sc), True)
        neg_h = (1.0 - s) * nh3 + s * nt3
        big_o[...] = neg_h
        start(big_o, negh_hbm, sem_o2)
        wait(big_o, negh_hbm, sem_o2)


def kernel(enc1_w, enc1_b, enc2_w, enc2_b, enc3_w, enc3_b, z_w, z_b,
           dec1_w, dec1_b, dec2_w, dec2_b, dec3_w, dec3_b, xbar_w, xbar_b,
           gnn1_w, gnn2_w, gnn3_w, gnn4_w, gnn5_w, cluster_layer, disc_weight,
           x, adj, perm):
    n, n_input = x.shape
    n_enc_3 = enc3_w.shape[1]
    n_z = z_w.shape[1]
    n_clusters = gnn5_w.shape[1]
    v = 1.0

    f32 = jnp.float32

    def b2(b):
        return b.reshape(1, -1)

    vmem_ops = [
        perm.astype(jnp.int32).reshape(n, 1),
        b2(enc1_b), b2(enc2_b), b2(enc3_b), b2(z_b),
        b2(dec1_b), b2(dec2_b), b2(dec3_b), b2(xbar_b),
        gnn5_w, cluster_layer,
    ]
    hbm_ops = [
        x, adj, enc1_w, enc2_w, enc3_w, z_w,
        dec1_w, dec2_w, dec3_w, xbar_w,
        gnn1_w, gnn2_w, gnn3_w, gnn4_w,
    ]

    in_specs = [_spec2(op.shape) for op in vmem_ops]
    in_specs += [pl.BlockSpec(memory_space=pl.ANY) for _ in hbm_ops]

    out_shapes = (
        jax.ShapeDtypeStruct((n, n_clusters), f32),
        jax.ShapeDtypeStruct((n, n_clusters), f32),
        jax.ShapeDtypeStruct((n, n_clusters), f32),
        jax.ShapeDtypeStruct((n, n_z), f32),
        jax.ShapeDtypeStruct((n, n_enc_3), f32),
        jax.ShapeDtypeStruct((n, n_enc_3), f32),
        jax.ShapeDtypeStruct((n, n_input), f32),
        jax.ShapeDtypeStruct((1, n_enc_3), f32),
    )
    out_specs = [pl.BlockSpec(memory_space=pl.ANY) for _ in out_shapes]

    scratch_shapes = [
        pltpu.VMEM(enc1_w.shape, f32),
        pltpu.VMEM(enc2_w.shape, f32),
        pltpu.VMEM(enc3_w.shape, f32),
        pltpu.VMEM(z_w.shape, f32),
        pltpu.VMEM(dec1_w.shape, f32),
        pltpu.VMEM(dec2_w.shape, f32),
        pltpu.VMEM(xbar_w.shape, f32),
        pltpu.VMEM(adj.shape, f32),
        pltpu.VMEM(x.shape, f32),
        pltpu.VMEM((n, n_enc_3), f32),
        pltpu.VMEM((n, n_z), f32),
        pltpu.VMEM((n, n_clusters), f32),
        pltpu.VMEM((n, n_clusters), f32),
        pltpu.VMEM((n, n_clusters), f32),
        pltpu.VMEM((1, n_enc_3), f32),
    ] + [pltpu.SemaphoreType.DMA(())] * 11

    outs = pl.pallas_call(
        functools.partial(_fused_kernel, v=v),
        grid=(2,),
        in_specs=in_specs,
        out_specs=out_specs,
        out_shape=out_shapes,
        scratch_shapes=scratch_shapes,
        compiler_params=pltpu.CompilerParams(
            dimension_semantics=("parallel",),
            vmem_limit_bytes=64 * 1024 * 1024,
        ),
    )(*vmem_ops, *hbm_ops)

    h, pred, q, z, out_h, neg_h, x_bar, summary = outs
    return h, out_h, neg_h, summary.reshape(-1), x_bar, q, pred, z


# speedup vs baseline: 1.3257x; 1.3257x over previous
import functools

import jax
import jax.numpy as jnp
from jax.experimental import pallas as pl
from jax.experimental.pallas import tpu as pltpu

_SIGMA = 0.5


def _spec2(shape):
    return pl.BlockSpec(shape, lambda i: (0, 0))


def _fused_kernel(
    permi_ref,
    e1b_ref, e2b_ref, e3b_ref, zb_ref,
    d1b_ref, d2b_ref, d3b_ref, xbb_ref,
    g5w_ref, clu_ref,
    x_hbm, adj_hbm, e1w_hbm, e2w_hbm, e3w_hbm, zw_hbm,
    d1w_hbm, d2w_hbm, d3w_hbm, xbw_hbm,
    g1w_hbm, g2w_hbm, g3w_hbm, g4w_hbm,
    h_hbm, pred_hbm, q_hbm, z_hbm, outh_hbm, negh_hbm, xbar_hbm, summ_hbm,
    sa, sb, sc, sd, se, sf, sg, adj_s, x_s,
    ho_s, po_s, qo_s, zo_s, oh_s, nh_s, xb_s, su_s,
    sem_a, sem_b, sem_c, sem_d, sem_e, sem_f, sem_g, sem_adj, sem_x,
    sem_o1, sem_o2,
    *, v,
):
    f32 = jnp.float32
    bf16 = jnp.bfloat16
    s = _SIGMA
    n = x_s.shape[0]

    def start(src, dst, sem):
        pltpu.make_async_copy(src, dst, sem).start()

    def wait(src, dst, sem):
        pltpu.make_async_copy(src, dst, sem).wait()

    start(e1w_hbm, sa, sem_a)
    start(x_hbm, x_s, sem_x)
    start(e2w_hbm, sb, sem_b)
    start(e3w_hbm, sc, sem_c)
    start(zw_hbm, sd, sem_d)
    start(adj_hbm, adj_s, sem_adj)

    def lin(xv, w_ref, b_ref, relu):
        y = jnp.dot(xv.astype(bf16), w_ref[...].astype(bf16),
                    preferred_element_type=f32)
        y = y + b_ref[...]
        return jnp.maximum(y, 0.0) if relu else y

    lane = jax.lax.broadcasted_iota(jnp.int32, (n, n), 1)
    pmat = (lane == permi_ref[...]).astype(bf16)

    wait(x_hbm, x_s, sem_x)
    xb = x_s[...].astype(bf16)

    wait(e1w_hbm, sa, sem_a)
    t1 = lin(xb, sa, e1b_ref, True)
    start(g1w_hbm, sa, sem_a)

    wait(e2w_hbm, sb, sem_b)
    t2 = lin(t1, sb, e2b_ref, True)
    start(g2w_hbm, sb, sem_b)

    wait(e3w_hbm, sc, sem_c)
    t3 = lin(t2, sc, e3b_ref, True)
    start(g3w_hbm, sc, sem_c)

    wait(zw_hbm, sd, sem_d)
    z = lin(t3, sd, zb_ref, False)
    start(g4w_hbm, sd, sem_d)
    start(d1w_hbm, se, sem_e)
    start(d2w_hbm, sf, sem_f)
    start(xbw_hbm, sg, sem_g)

    zo_s[...] = z
    start(zo_s, z_hbm, sem_o1)

    tcat = jnp.concatenate(
        [t1.astype(bf16), t2.astype(bf16), t3.astype(bf16)], axis=1)
    ncat = jnp.dot(pmat, tcat, preferred_element_type=f32).astype(bf16)
    c1 = t1.shape[1]
    c2 = c1 + t2.shape[1]
    nt1 = ncat[:, :c1]
    nt2 = ncat[:, c1:c2]
    nt3 = ncat[:, c2:]

    wait(adj_hbm, adj_s, sem_adj)
    adjb = adj_s[...].astype(bf16)

    def agg(sup, relu):
        y = jnp.dot(adjb, sup.astype(bf16), preferred_element_type=f32)
        return jnp.maximum(y, 0.0) if relu else y

    def mm(xv, w_ref):
        return jnp.dot(xv.astype(bf16), w_ref[...].astype(bf16),
                       preferred_element_type=f32)

    wait(g1w_hbm, sa, sem_a)
    h1 = agg(mm(xb, sa), True)

    wait(g2w_hbm, sb, sem_b)
    h2 = agg(mm((1.0 - s) * h1 + s * t1, sb), True)
    nh2 = agg(mm((1.0 - s) * h1 + s * nt1, sb), True)
    start(d3w_hbm, sb, sem_b)

    wait(g3w_hbm, sc, sem_c)
    h3 = agg(mm((1.0 - s) * h2 + s * t2, sc), True)
    out_h = (1.0 - s) * h3 + s * t3
    oh_s[...] = out_h
    start(oh_s, outh_hbm, sem_o1)

    wait(g4w_hbm, sd, sem_d)
    h4 = agg(mm(out_h, sd), True)
    h5 = agg(jnp.dot(((1.0 - s) * h4 + s * z).astype(bf16),
                     g5w_ref[...].astype(bf16),
                     preferred_element_type=f32), False)

    m = jnp.max(h5, axis=1, keepdims=True)
    e = jnp.exp(h5 - m)
    pred = e * pl.reciprocal(jnp.sum(e, axis=1, keepdims=True), approx=True)

    mean = jnp.mean(out_h, axis=0, keepdims=True)
    en = jnp.exp(-jnp.abs(mean))
    r = pl.reciprocal(1.0 + en, approx=True)
    su_s[...] = jnp.where(mean >= 0.0, r, en * r)

    c = clu_ref[...]
    z2 = jnp.sum(z * z, axis=1, keepdims=True)
    c2 = jnp.sum(c * c, axis=1, keepdims=True).T
    zc = jnp.dot(z, c.T, preferred_element_type=f32)
    dist = jnp.maximum(z2 + c2 - 2.0 * zc, 0.0)
    q = pl.reciprocal(1.0 + dist * (1.0 / v), approx=True)
    if v != 1.0:
        q = q ** ((v + 1.0) / 2.0)
    q = q * pl.reciprocal(jnp.sum(q, axis=1, keepdims=True), approx=True)

    ho_s[...] = h5
    po_s[...] = pred
    qo_s[...] = q
    start(ho_s, h_hbm, sem_o1)
    start(po_s, pred_hbm, sem_o1)
    start(qo_s, q_hbm, sem_o1)
    start(su_s, summ_hbm, sem_o1)

    nh3 = agg(mm((1.0 - s) * nh2 + s * nt2, sc), True)
    neg_h = (1.0 - s) * nh3 + s * nt3
    nh_s[...] = neg_h
    start(nh_s, negh_hbm, sem_o1)

    wait(d1w_hbm, se, sem_e)
    d1 = lin(z, se, d1b_ref, True)
    wait(d2w_hbm, sf, sem_f)
    d2 = lin(d1, sf, d2b_ref, True)
    wait(d3w_hbm, sb, sem_b)
    d3 = lin(d2, sb, d3b_ref, True)
    wait(xbw_hbm, sg, sem_g)
    xb_s[...] = lin(d3, sg, xbb_ref, False)
    start(xb_s, xbar_hbm, sem_o2)

    wait(zo_s, z_hbm, sem_o1)
    wait(oh_s, outh_hbm, sem_o1)
    wait(nh_s, negh_hbm, sem_o1)
    wait(ho_s, h_hbm, sem_o1)
    wait(po_s, pred_hbm, sem_o1)
    wait(qo_s, q_hbm, sem_o1)
    wait(su_s, summ_hbm, sem_o1)
    wait(xb_s, xbar_hbm, sem_o2)


def kernel(enc1_w, enc1_b, enc2_w, enc2_b, enc3_w, enc3_b, z_w, z_b,
           dec1_w, dec1_b, dec2_w, dec2_b, dec3_w, dec3_b, xbar_w, xbar_b,
           gnn1_w, gnn2_w, gnn3_w, gnn4_w, gnn5_w, cluster_layer, disc_weight,
           x, adj, perm):
    n, n_input = x.shape
    n_enc_3 = enc3_w.shape[1]
    n_z = z_w.shape[1]
    n_clusters = gnn5_w.shape[1]
    v = 1.0

    f32 = jnp.float32

    def b2(b):
        return b.reshape(1, -1)

    vmem_ops = [
        perm.astype(jnp.int32).reshape(n, 1),
        b2(enc1_b), b2(enc2_b), b2(enc3_b), b2(z_b),
        b2(dec1_b), b2(dec2_b), b2(dec3_b), b2(xbar_b),
        gnn5_w, cluster_layer,
    ]
    hbm_ops = [
        x, adj, enc1_w, enc2_w, enc3_w, z_w,
        dec1_w, dec2_w, dec3_w, xbar_w,
        gnn1_w, gnn2_w, gnn3_w, gnn4_w,
    ]

    in_specs = [_spec2(op.shape) for op in vmem_ops]
    in_specs += [pl.BlockSpec(memory_space=pl.ANY) for _ in hbm_ops]

    out_shapes = (
        jax.ShapeDtypeStruct((n, n_clusters), f32),
        jax.ShapeDtypeStruct((n, n_clusters), f32),
        jax.ShapeDtypeStruct((n, n_clusters), f32),
        jax.ShapeDtypeStruct((n, n_z), f32),
        jax.ShapeDtypeStruct((n, n_enc_3), f32),
        jax.ShapeDtypeStruct((n, n_enc_3), f32),
        jax.ShapeDtypeStruct((n, n_input), f32),
        jax.ShapeDtypeStruct((1, n_enc_3), f32),
    )
    out_specs = [pl.BlockSpec(memory_space=pl.ANY) for _ in out_shapes]

    scratch_shapes = [
        pltpu.VMEM(enc1_w.shape, f32),
        pltpu.VMEM(enc2_w.shape, f32),
        pltpu.VMEM(enc3_w.shape, f32),
        pltpu.VMEM(z_w.shape, f32),
        pltpu.VMEM(dec1_w.shape, f32),
        pltpu.VMEM(dec2_w.shape, f32),
        pltpu.VMEM(xbar_w.shape, f32),
        pltpu.VMEM(adj.shape, f32),
        pltpu.VMEM(x.shape, f32),
    ] + [pltpu.VMEM(sh.shape, f32) for sh in out_shapes] \
      + [pltpu.SemaphoreType.DMA(())] * 11

    outs = pl.pallas_call(
        functools.partial(_fused_kernel, v=v),
        grid=(1,),
        in_specs=in_specs,
        out_specs=out_specs,
        out_shape=out_shapes,
        scratch_shapes=scratch_shapes,
        compiler_params=pltpu.CompilerParams(
            dimension_semantics=("arbitrary",),
            vmem_limit_bytes=64 * 1024 * 1024,
        ),
    )(*vmem_ops, *hbm_ops)

    h, pred, q, z, out_h, neg_h, x_bar, summary = outs
    return h, out_h, neg_h, summary.reshape(-1), x_bar, q, pred, z


# speedup vs baseline: 1.3307x; 1.0038x over previous
import functools

import jax
import jax.numpy as jnp
from jax.experimental import pallas as pl
from jax.experimental.pallas import tpu as pltpu

_SIGMA = 0.5


def _spec2(shape):
    return pl.BlockSpec(shape, lambda i: (0, 0))


def _fused_kernel(
    permi_ref,
    e1b_ref, e2b_ref, e3b_ref, zb_ref,
    d1b_ref, d2b_ref, d3b_ref, xbb_ref,
    g5w_ref, clu_ref,
    x_hbm, adj_hbm, e1w_hbm, e2w_hbm, e3w_hbm, zw_hbm,
    d1w_hbm, d2w_hbm, d3w_hbm, xbw_hbm,
    g1w_hbm, g2w_hbm, g3w_hbm, g4w_hbm,
    h_hbm, pred_hbm, q_hbm, z_hbm, outh_hbm, negh_hbm, xbar_hbm, summ_hbm,
    sa, sb, sc, sd, se, sf, sg, adj_s, x_s,
    ho_s, po_s, qo_s, zo_s, oh_s, nh_s, xb_s, su_s,
    sem_a, sem_b, sem_c, sem_d, sem_e, sem_f, sem_g, sem_adj, sem_x,
    sem_o1, sem_o2,
    *, v,
):
    f32 = jnp.float32
    bf16 = jnp.bfloat16
    s = _SIGMA
    n = x_s.shape[0]

    def start(src, dst, sem):
        pltpu.make_async_copy(src, dst, sem).start()

    def wait(src, dst, sem):
        pltpu.make_async_copy(src, dst, sem).wait()

    kh = x_s.shape[1] // 2
    start(x_hbm.at[:, :kh], x_s.at[:, :kh], sem_x)
    start(e1w_hbm.at[:kh], sa.at[:kh], sem_a)
    start(x_hbm.at[:, kh:], x_s.at[:, kh:], sem_x)
    start(e1w_hbm.at[kh:], sa.at[kh:], sem_a)
    start(e2w_hbm, sb, sem_b)
    start(e3w_hbm, sc, sem_c)
    start(zw_hbm, sd, sem_d)
    start(adj_hbm, adj_s, sem_adj)

    def lin(xv, w_ref, b_ref, relu):
        y = jnp.dot(xv.astype(bf16), w_ref[...].astype(bf16),
                    preferred_element_type=f32)
        y = y + b_ref[...]
        return jnp.maximum(y, 0.0) if relu else y

    lane = jax.lax.broadcasted_iota(jnp.int32, (n, n), 1)
    pmat = (lane == permi_ref[...]).astype(bf16)

    wait(x_hbm.at[:, :kh], x_s.at[:, :kh], sem_x)
    wait(e1w_hbm.at[:kh], sa.at[:kh], sem_a)
    acc1 = jnp.dot(x_s[:, :kh].astype(bf16), sa[:kh].astype(bf16),
                   preferred_element_type=f32)
    wait(x_hbm.at[:, kh:], x_s.at[:, kh:], sem_x)
    wait(e1w_hbm.at[kh:], sa.at[kh:], sem_a)
    xb = x_s[...].astype(bf16)
    acc2 = jnp.dot(xb[:, kh:], sa[kh:].astype(bf16),
                   preferred_element_type=f32)
    t1 = jnp.maximum(acc1 + acc2 + e1b_ref[...], 0.0)
    start(g1w_hbm, sa, sem_a)

    wait(e2w_hbm, sb, sem_b)
    t2 = lin(t1, sb, e2b_ref, True)
    start(g2w_hbm, sb, sem_b)

    wait(e3w_hbm, sc, sem_c)
    t3 = lin(t2, sc, e3b_ref, True)
    start(g3w_hbm, sc, sem_c)

    wait(zw_hbm, sd, sem_d)
    z = lin(t3, sd, zb_ref, False)
    start(g4w_hbm, sd, sem_d)
    start(d1w_hbm, se, sem_e)
    start(d2w_hbm, sf, sem_f)
    start(xbw_hbm, sg, sem_g)

    zo_s[...] = z
    start(zo_s, z_hbm, sem_o1)

    tcat = jnp.concatenate(
        [t1.astype(bf16), t2.astype(bf16), t3.astype(bf16)], axis=1)
    ncat = jnp.dot(pmat, tcat, preferred_element_type=f32).astype(bf16)
    c1 = t1.shape[1]
    c2 = c1 + t2.shape[1]
    nt1 = ncat[:, :c1]
    nt2 = ncat[:, c1:c2]
    nt3 = ncat[:, c2:]

    wait(adj_hbm, adj_s, sem_adj)
    adjb = adj_s[...].astype(bf16)

    def agg(sup, relu):
        y = jnp.dot(adjb, sup.astype(bf16), preferred_element_type=f32)
        return jnp.maximum(y, 0.0) if relu else y

    def mm(xv, w_ref):
        return jnp.dot(xv.astype(bf16), w_ref[...].astype(bf16),
                       preferred_element_type=f32)

    wait(g1w_hbm, sa, sem_a)
    h1 = agg(mm(xb, sa), True)

    wait(g2w_hbm, sb, sem_b)
    h2 = agg(mm((1.0 - s) * h1 + s * t1, sb), True)
    nh2 = agg(mm((1.0 - s) * h1 + s * nt1, sb), True)
    start(d3w_hbm, sb, sem_b)

    wait(g3w_hbm, sc, sem_c)
    h3 = agg(mm((1.0 - s) * h2 + s * t2, sc), True)
    out_h = (1.0 - s) * h3 + s * t3
    oh_s[...] = out_h
    start(oh_s, outh_hbm, sem_o1)

    wait(g4w_hbm, sd, sem_d)
    h4 = agg(mm(out_h, sd), True)
    h5 = agg(jnp.dot(((1.0 - s) * h4 + s * z).astype(bf16),
                     g5w_ref[...].astype(bf16),
                     preferred_element_type=f32), False)

    m = jnp.max(h5, axis=1, keepdims=True)
    e = jnp.exp(h5 - m)
    pred = e * pl.reciprocal(jnp.sum(e, axis=1, keepdims=True), approx=True)

    mean = jnp.mean(out_h, axis=0, keepdims=True)
    en = jnp.exp(-jnp.abs(mean))
    r = pl.reciprocal(1.0 + en, approx=True)
    su_s[...] = jnp.where(mean >= 0.0, r, en * r)

    c = clu_ref[...]
    z2 = jnp.sum(z * z, axis=1, keepdims=True)
    c2 = jnp.sum(c * c, axis=1, keepdims=True).T
    zc = jnp.dot(z, c.T, preferred_element_type=f32)
    dist = jnp.maximum(z2 + c2 - 2.0 * zc, 0.0)
    q = pl.reciprocal(1.0 + dist * (1.0 / v), approx=True)
    if v != 1.0:
        q = q ** ((v + 1.0) / 2.0)
    q = q * pl.reciprocal(jnp.sum(q, axis=1, keepdims=True), approx=True)

    ho_s[...] = h5
    po_s[...] = pred
    qo_s[...] = q
    start(ho_s, h_hbm, sem_o1)
    start(po_s, pred_hbm, sem_o1)
    start(qo_s, q_hbm, sem_o1)
    start(su_s, summ_hbm, sem_o1)

    nh3 = agg(mm((1.0 - s) * nh2 + s * nt2, sc), True)
    neg_h = (1.0 - s) * nh3 + s * nt3
    nh_s[...] = neg_h
    start(nh_s, negh_hbm, sem_o1)

    wait(d1w_hbm, se, sem_e)
    d1 = lin(z, se, d1b_ref, True)
    wait(d2w_hbm, sf, sem_f)
    d2 = lin(d1, sf, d2b_ref, True)
    wait(d3w_hbm, sb, sem_b)
    d3 = lin(d2, sb, d3b_ref, True)
    wait(xbw_hbm, sg, sem_g)
    xb_s[...] = lin(d3, sg, xbb_ref, False)
    start(xb_s, xbar_hbm, sem_o2)

    wait(zo_s, z_hbm, sem_o1)
    wait(oh_s, outh_hbm, sem_o1)
    wait(nh_s, negh_hbm, sem_o1)
    wait(ho_s, h_hbm, sem_o1)
    wait(po_s, pred_hbm, sem_o1)
    wait(qo_s, q_hbm, sem_o1)
    wait(su_s, summ_hbm, sem_o1)
    wait(xb_s, xbar_hbm, sem_o2)


def kernel(enc1_w, enc1_b, enc2_w, enc2_b, enc3_w, enc3_b, z_w, z_b,
           dec1_w, dec1_b, dec2_w, dec2_b, dec3_w, dec3_b, xbar_w, xbar_b,
           gnn1_w, gnn2_w, gnn3_w, gnn4_w, gnn5_w, cluster_layer, disc_weight,
           x, adj, perm):
    n, n_input = x.shape
    n_enc_3 = enc3_w.shape[1]
    n_z = z_w.shape[1]
    n_clusters = gnn5_w.shape[1]
    v = 1.0

    f32 = jnp.float32

    def b2(b):
        return b.reshape(1, -1)

    vmem_ops = [
        perm.astype(jnp.int32).reshape(n, 1),
        b2(enc1_b), b2(enc2_b), b2(enc3_b), b2(z_b),
        b2(dec1_b), b2(dec2_b), b2(dec3_b), b2(xbar_b),
        gnn5_w, cluster_layer,
    ]
    hbm_ops = [
        x, adj, enc1_w, enc2_w, enc3_w, z_w,
        dec1_w, dec2_w, dec3_w, xbar_w,
        gnn1_w, gnn2_w, gnn3_w, gnn4_w,
    ]

    in_specs = [_spec2(op.shape) for op in vmem_ops]
    in_specs += [pl.BlockSpec(memory_space=pl.ANY) for _ in hbm_ops]

    out_shapes = (
        jax.ShapeDtypeStruct((n, n_clusters), f32),
        jax.ShapeDtypeStruct((n, n_clusters), f32),
        jax.ShapeDtypeStruct((n, n_clusters), f32),
        jax.ShapeDtypeStruct((n, n_z), f32),
        jax.ShapeDtypeStruct((n, n_enc_3), f32),
        jax.ShapeDtypeStruct((n, n_enc_3), f32),
        jax.ShapeDtypeStruct((n, n_input), f32),
        jax.ShapeDtypeStruct((1, n_enc_3), f32),
    )
    out_specs = [pl.BlockSpec(memory_space=pl.ANY) for _ in out_shapes]

    scratch_shapes = [
        pltpu.VMEM(enc1_w.shape, f32),
        pltpu.VMEM(enc2_w.shape, f32),
        pltpu.VMEM(enc3_w.shape, f32),
        pltpu.VMEM(z_w.shape, f32),
        pltpu.VMEM(dec1_w.shape, f32),
        pltpu.VMEM(dec2_w.shape, f32),
        pltpu.VMEM(xbar_w.shape, f32),
        pltpu.VMEM(adj.shape, f32),
        pltpu.VMEM(x.shape, f32),
    ] + [pltpu.VMEM(sh.shape, f32) for sh in out_shapes] \
      + [pltpu.SemaphoreType.DMA(())] * 11

    outs = pl.pallas_call(
        functools.partial(_fused_kernel, v=v),
        grid=(1,),
        in_specs=in_specs,
        out_specs=out_specs,
        out_shape=out_shapes,
        scratch_shapes=scratch_shapes,
        compiler_params=pltpu.CompilerParams(
            dimension_semantics=("arbitrary",),
            vmem_limit_bytes=64 * 1024 * 1024,
        ),
    )(*vmem_ops, *hbm_ops)

    h, pred, q, z, out_h, neg_h, x_bar, summary = outs
    return h, out_h, neg_h, summary.reshape(-1), x_bar, q, pred, z
